# Initial kernel scaffold; baseline (speedup 1.0000x reference)
import jax
import jax.numpy as jnp
from jax.experimental import pallas as pl


def kernel(x):
    raise NotImplementedError("write your pallas kernel here")


# speedup vs baseline: 73.6179x; 73.6179x over previous
import functools

import jax
import jax.numpy as jnp
from jax import lax
from jax.experimental import pallas as pl
from jax.experimental.pallas import tpu as pltpu
from jax.experimental.pallas import tpu_sc as plsc

NC = 2
NS = 16
L = 16
NW = NC * NS
BINS = 256


def _sc_partial_hists(xf, ch, n_per_ch):
    chunk = n_per_ch // NW
    mesh = plsc.VectorSubcoreMesh(
        core_axis_name="c", subcore_axis_name="s", num_cores=NC, num_subcores=NS
    )

    @functools.partial(
        pl.kernel,
        out_type=jax.ShapeDtypeStruct((ch, NW, BINS), jnp.float32),
        mesh=mesh,
        compiler_params=pltpu.CompilerParams(needs_layout_passes=False),
        scratch_types=[
            pltpu.VMEM((chunk,), jnp.float32),
            pltpu.VMEM((L * BINS,), jnp.float32),
            pltpu.VMEM((BINS,), jnp.float32),
        ],
    )
    def hist_kernel(x_hbm, out_hbm, buf, sub, red):
        wid = lax.axis_index("s") * NC + lax.axis_index("c")
        lanebase = lax.broadcasted_iota(jnp.int32, (L,), 0) * BINS
        ones = jnp.ones((L,), jnp.float32)
        zeros = jnp.zeros((L,), jnp.float32)

        def zero_body(i, carry):
            sub[pl.ds(i * L, L)] = zeros
            return carry

        lax.fori_loop(0, (L * BINS) // L, zero_body, None)

        def ch_body(c, carry):
            start = c * n_per_ch + wid * chunk
            pltpu.sync_copy(x_hbm.at[pl.ds(start, chunk)], buf)

            def h_body(i, hcarry):
                v = buf[pl.ds(i * L, L)]
                idx = (v * 256.0).astype(jnp.int32)
                plsc.addupdate_scatter(sub, [lanebase + idx], ones)
                return hcarry

            lax.fori_loop(0, chunk // L, h_body, None)

            def r_body(j, rcarry):
                acc = sub[pl.ds(j * L, L)]
                sub[pl.ds(j * L, L)] = zeros
                for r in range(1, L):
                    off = r * BINS + j * L
                    acc = acc + sub[pl.ds(off, L)]
                    sub[pl.ds(off, L)] = zeros
                red[pl.ds(j * L, L)] = acc
                return rcarry

            lax.fori_loop(0, BINS // L, r_body, None)

            pltpu.sync_copy(red, out_hbm.at[c, wid])
            return carry

        lax.fori_loop(0, ch, ch_body, None)

    return hist_kernel(xf)


def _tc_expand(partials, ch, out_h, out_w):
    def body(p_ref, o_ref):
        h_row = jnp.sum(p_ref[0], axis=0, keepdims=True)
        yi = lax.broadcasted_iota(jnp.int32, (out_h, BINS), 0).astype(jnp.float32)
        ki = lax.broadcasted_iota(jnp.int32, (out_h, BINS), 1).astype(jnp.float32)
        ys = jnp.maximum(yi * (BINS / out_h) + (0.5 * BINS / out_h - 0.5), 0.0)
        y0 = jnp.floor(ys)
        wy = ys - y0
        y1 = jnp.minimum(y0 + 1.0, float(BINS - 1))
        stencil = (jnp.where(ki == y0, 1.0 - wy, 0.0)
                   + jnp.where(ki == y1, wy, 0.0))
        vals = jnp.sum(stencil * h_row, axis=1, keepdims=True)
        o_ref[0] = jnp.broadcast_to(vals, (out_h, out_w))

    return pl.pallas_call(
        body,
        grid=(ch,),
        in_specs=[pl.BlockSpec((1, NW, BINS), lambda c: (c, 0, 0))],
        out_specs=pl.BlockSpec((1, out_h, out_w), lambda c: (c, 0, 0)),
        out_shape=jax.ShapeDtypeStruct((ch, out_h, out_w), jnp.float32),
    )(partials)


def kernel(x):
    b, c, h, w = x.shape
    ch = b * c
    n_per_ch = h * w
    xf = x.reshape(-1)
    partials = _sc_partial_hists(xf, ch, n_per_ch)
    out = _tc_expand(partials, ch, h, w)
    return out.reshape(b, c, h, w)


# speedup vs baseline: 169.7182x; 2.3054x over previous
import functools

import jax
import jax.numpy as jnp
from jax import lax
from jax.experimental import pallas as pl
from jax.experimental.pallas import tpu as pltpu
from jax.experimental.pallas import tpu_sc as plsc

NC = 2
NS = 16
L = 16
NW = NC * NS
BINS = 256


def _sc_partial_hists(xf, ch, n_per_ch):
    chunk = n_per_ch // NW
    mesh = plsc.VectorSubcoreMesh(
        core_axis_name="c", subcore_axis_name="s", num_cores=NC, num_subcores=NS
    )

    @functools.partial(
        pl.kernel,
        out_type=jax.ShapeDtypeStruct((NW, ch * BINS), jnp.float32),
        mesh=mesh,
        compiler_params=pltpu.CompilerParams(needs_layout_passes=False),
        scratch_types=[
            pltpu.VMEM((chunk,), jnp.float32),
            pltpu.VMEM((chunk,), jnp.float32),
            pltpu.VMEM((L * BINS,), jnp.float32),
            pltpu.VMEM((ch * BINS,), jnp.float32),
            pltpu.SemaphoreType.DMA,
            pltpu.SemaphoreType.DMA,
        ],
    )
    def hist_kernel(x_hbm, out_hbm, buf_a, buf_b, sub, red, sem_a, sem_b):
        wid = lax.axis_index("s") * NC + lax.axis_index("c")
        lanebase = lax.broadcasted_iota(jnp.int32, (L,), 0) * BINS
        ones = jnp.ones((L,), jnp.float32)
        zeros = jnp.zeros((L,), jnp.float32)
        bufs = (buf_a, buf_b)
        sems = (sem_a, sem_b)

        @plsc.parallel_loop(0, L * BINS, step=L, unroll=4)
        def zero_body(i):
            sub[pl.ds(i, L)] = zeros

        def issue(c):
            start = c * n_per_ch + wid * chunk
            return pltpu.async_copy(
                x_hbm.at[pl.ds(start, chunk)], bufs[c % 2], sems[c % 2]
            )

        copies = {0: issue(0)}
        for c in range(ch):
            if c + 1 < ch:
                copies[c + 1] = issue(c + 1)
            copies[c].wait()
            buf = bufs[c % 2]

            @plsc.parallel_loop(0, chunk, step=L, unroll=8)
            def h_body(i):
                v = buf[pl.ds(i, L)]
                idx = (v * 256.0).astype(jnp.int32)
                plsc.addupdate_scatter(sub, [lanebase + idx], ones)

            @plsc.parallel_loop(0, BINS, step=L, unroll=2)
            def r_body(j):
                vs = []
                for r in range(L):
                    off = r * BINS + j
                    vs.append(sub[pl.ds(off, L)])
                    sub[pl.ds(off, L)] = zeros
                while len(vs) > 1:
                    vs = [a + b for a, b in zip(vs[::2], vs[1::2])]
                red[pl.ds(c * BINS + j, L)] = vs[0]

        pltpu.sync_copy(red, out_hbm.at[wid])

    return hist_kernel(xf)


def _tc_expand(partials, ch, out_h, out_w):
    def body(p_ref, o_ref):
        h_row = jnp.sum(p_ref[...], axis=0, keepdims=True)
        yi = lax.broadcasted_iota(jnp.int32, (out_h, BINS), 0).astype(jnp.float32)
        ki = lax.broadcasted_iota(jnp.int32, (out_h, BINS), 1).astype(jnp.float32)
        ys = jnp.maximum(yi * (BINS / out_h) + (0.5 * BINS / out_h - 0.5), 0.0)
        y0 = jnp.floor(ys)
        wy = ys - y0
        y1 = jnp.minimum(y0 + 1.0, float(BINS - 1))
        stencil = (jnp.where(ki == y0, 1.0 - wy, 0.0)
                   + jnp.where(ki == y1, wy, 0.0))
        vals = jnp.sum(stencil * h_row, axis=1, keepdims=True)
        o_ref[0] = jnp.broadcast_to(vals, (out_h, out_w))

    return pl.pallas_call(
        body,
        grid=(ch,),
        in_specs=[pl.BlockSpec((NW, BINS), lambda c: (0, c))],
        out_specs=pl.BlockSpec((1, out_h, out_w), lambda c: (c, 0, 0)),
        out_shape=jax.ShapeDtypeStruct((ch, out_h, out_w), jnp.float32),
    )(partials)


def kernel(x):
    b, c, h, w = x.shape
    ch = b * c
    n_per_ch = h * w
    xf = x.reshape(-1)
    partials = _sc_partial_hists(xf, ch, n_per_ch)
    out = _tc_expand(partials, ch, h, w)
    return out.reshape(b, c, h, w)
